# Initial kernel scaffold; baseline (speedup 1.0000x reference)
import jax
import jax.numpy as jnp
from jax.experimental import pallas as pl


def kernel(hn, he, fe, fes, norm, edge_index, We1, be1, We2, Wf1, Wf2, Wu1, bu1, Wu2, Wn1, bn1, Wn2):
    raise NotImplementedError("write your pallas kernel here")


# speedup vs baseline: 1.1703x; 1.1703x over previous
import functools

import numpy as np
import jax
import jax.numpy as jnp
from jax import lax
from jax.experimental import pallas as pl
from jax.experimental.pallas import tpu as pltpu

_D = 128
_DSH = 9
_EB = 16
_FCH = 16
_HID = 4 * _D


def _edge_block_kernel(he_ref, hs_ref, hd_ref, fe_ref, fes_ref, norm_ref,
                       We1_ref, be1_ref, We2_ref, Wf1_ref, Wf2_ref,
                       Wu1_ref, bu1_ref, Wu2_ref, S_ref,
                       he_out_ref, scaled_ref):
    he = he_ref[...]
    hs = hs_ref[...]
    hd = hd_ref[...]
    x = jnp.concatenate([he, hs, hd], axis=1)
    a = jnp.dot(x, We1_ref[...], preferred_element_type=jnp.float32) + be1_ref[...]
    av = a * jax.nn.sigmoid(a)
    v = jnp.dot(av, We2_ref[...], preferred_element_type=jnp.float32)
    r = jnp.maximum(
        jnp.dot(fes_ref[...], Wf1_ref[...], preferred_element_type=jnp.float32)
        * (1.0 / np.sqrt(_EB)), 0.0)
    w = jnp.dot(r, Wf2_ref[...], preferred_element_type=jnp.float32) \
        * (1.0 / np.sqrt(_FCH))
    d = jnp.dot(v * fe_ref[...], S_ref[...],
                preferred_element_type=jnp.float32)
    tp = (w[:, 0:_D] * d[:, 0:1]
          + w[:, _D:2 * _D] * d[:, 1:2]
          + w[:, 2 * _D:3 * _D] * d[:, 2:3])
    u = jnp.concatenate([tp, hs, hd], axis=1)
    b = jnp.dot(u, Wu1_ref[...], preferred_element_type=jnp.float32) + bu1_ref[...]
    bv = b * jax.nn.sigmoid(b)
    he_new = he + jnp.dot(bv, Wu2_ref[...], preferred_element_type=jnp.float32)
    he_out_ref[...] = he_new
    scaled_ref[...] = he_new * norm_ref[...]


def _edge_stage(he, hs, hd, fe, fes, norm2d, We1, be1, We2, Wf1, Wf2,
                Wu1, bu1, Wu2, S):
    E = he.shape[0]
    BE = 2560 if E % 2560 == 0 else 512
    grid = (E // BE,)
    full = lambda shape: pl.BlockSpec(shape, lambda i: (0, 0))
    blk = lambda w: pl.BlockSpec((BE, w), lambda i: (i, 0))
    return pl.pallas_call(
        _edge_block_kernel,
        grid=grid,
        in_specs=[
            blk(_D), blk(_D), blk(_D), blk(_DSH), blk(_EB), blk(1),
            full(We1.shape), full((1, _HID)), full(We2.shape),
            full(Wf1.shape), full(Wf2.shape),
            full(Wu1.shape), full((1, _HID)), full(Wu2.shape),
            full(S.shape),
        ],
        out_specs=[blk(_D), blk(_D)],
        out_shape=[
            jax.ShapeDtypeStruct((E, _D), jnp.float32),
            jax.ShapeDtypeStruct((E, _D), jnp.float32),
        ],
    )(he, hs, hd, fe, fes, norm2d, We1, be1.reshape(1, -1), We2, Wf1, Wf2,
      Wu1, bu1.reshape(1, -1), Wu2, S)


def _node_block_kernel(hn_ref, nf_ref, Wn1_ref, bn1_ref, Wn2_ref, out_ref):
    hn = hn_ref[...]
    x = jnp.concatenate([hn, nf_ref[...]], axis=1)
    a = jnp.dot(x, Wn1_ref[...], preferred_element_type=jnp.float32) + bn1_ref[...]
    av = a * jax.nn.sigmoid(a)
    out_ref[...] = hn + jnp.dot(av, Wn2_ref[...], preferred_element_type=jnp.float32)


def _node_stage(hn, node_ftr, Wn1, bn1, Wn2):
    N = hn.shape[0]
    BN = 2000 if N % 2000 == 0 else N
    grid = (N // BN,)
    full = lambda shape: pl.BlockSpec(shape, lambda i: (0, 0))
    blk = lambda w: pl.BlockSpec((BN, w), lambda i: (i, 0))
    return pl.pallas_call(
        _node_block_kernel,
        grid=grid,
        in_specs=[blk(_D), blk(_D), full(Wn1.shape), full((1, _HID)),
                  full(Wn2.shape)],
        out_specs=blk(_D),
        out_shape=jax.ShapeDtypeStruct((N, _D), jnp.float32),
    )(hn, node_ftr, Wn1, bn1.reshape(1, -1), Wn2)


def kernel(hn, he, fe, fes, norm, edge_index, We1, be1, We2, Wf1, Wf2,
           Wu1, bu1, Wu2, Wn1, bn1, Wn2):
    src = edge_index[0]
    dst = edge_index[1]
    S = np.zeros((_DSH, 3), dtype=np.float32)
    S[0, 0] = 1.0
    S[1:4, 1] = 1.0 / np.sqrt(3.0)
    S[4:9, 2] = 1.0 / np.sqrt(5.0)
    S = jnp.asarray(S / np.sqrt(3.0))

    hs = jnp.take(hn, src, axis=0)
    hd = jnp.take(hn, dst, axis=0)

    he_new, scaled = _edge_stage(he, hs, hd, fe, fes, norm.reshape(-1, 1),
                                 We1, be1, We2, Wf1, Wf2, Wu1, bu1, Wu2, S)

    node_ftr = jax.ops.segment_sum(scaled, dst, num_segments=hn.shape[0])

    hn_new = _node_stage(hn, node_ftr, Wn1, bn1, Wn2)
    return hn_new, he_new


# speedup vs baseline: 2.8390x; 2.4259x over previous
import functools

import numpy as np
import jax
import jax.numpy as jnp
from jax import lax
from jax.experimental import pallas as pl
from jax.experimental.pallas import tpu as pltpu
from jax.experimental.pallas import tpu_sc as plsc

_NC = 2
_NS = 16
_NW = _NC * _NS

_D = 128
_DSH = 9
_EB = 16
_FCH = 16
_HID = 4 * _D


_CH = 80


def _gather_stage(hn, src, dst):
    N, D = hn.shape
    E = src.shape[0]
    per_w = E // _NW
    n_ch = per_w // _CH
    assert per_w * _NW == E and n_ch * _CH == per_w
    mesh = plsc.VectorSubcoreMesh(core_axis_name="c", subcore_axis_name="s")

    @functools.partial(
        pl.kernel, mesh=mesh,
        out_type=[jax.ShapeDtypeStruct((E, D), jnp.float32),
                  jax.ShapeDtypeStruct((E, D), jnp.float32)],
        scratch_types=[
            pltpu.VMEM((per_w,), jnp.int32),
            pltpu.VMEM((_CH, D), jnp.float32),
            pltpu.VMEM((_CH, D), jnp.float32),
            pltpu.SemaphoreType.DMA,
            pltpu.SemaphoreType.DMA,
        ],
    )
    def k(hn_hbm, src_hbm, dst_hbm, hs_hbm, hd_hbm,
          idx_v, buf0, buf1, gsem, osem):
        wid = lax.axis_index("s") * _NC + lax.axis_index("c")
        base = wid * per_w
        for idx_hbm, out_hbm in ((src_hbm, hs_hbm), (dst_hbm, hd_hbm)):
            pltpu.sync_copy(idx_hbm.at[pl.ds(base, per_w)], idx_v)

            def body(i, _, out_hbm=out_hbm):
                g = pltpu.async_copy(
                    hn_hbm.at[idx_v.at[pl.ds(i * _CH, _CH)]], buf0, gsem)
                g.wait()
                pltpu.sync_copy(buf0, out_hbm.at[pl.ds(base + i * _CH, _CH)])
                return 0

            lax.fori_loop(0, n_ch, body, 0)

    return k(hn, src, dst)


def _scatter_stage(scaled, dst3, zeros, N_pad):
    E, D = scaled.shape
    per_w = E // _NW
    n_ch = per_w // _CH
    rows_per_tile = N_pad // _NS
    assert rows_per_tile * _NS == N_pad and rows_per_tile % 8 == 0
    mesh = plsc.VectorSubcoreMesh(core_axis_name="c", subcore_axis_name="s")

    @functools.partial(
        pl.kernel, mesh=mesh,
        out_type=jax.ShapeDtypeStruct((_NC, N_pad, D), jnp.float32),
        scratch_types=[
            pltpu.VMEM((n_ch, _CH), jnp.int32),
            pltpu.VMEM((_CH, D), jnp.float32),
            pltpu.VMEM_SHARED((N_pad, D), jnp.float32),
        ],
    )
    def k(scaled_hbm, dst3_hbm, zeros_hbm, out_hbm, idx_v, rows_v, acc_sh):
        cid = lax.axis_index("c")
        sid = lax.axis_index("s")
        wid = sid * _NC + cid
        base = wid * per_w
        row0 = sid * rows_per_tile
        pltpu.sync_copy(zeros_hbm.at[pl.ds(row0, rows_per_tile)],
                        acc_sh.at[pl.ds(row0, rows_per_tile)])
        pltpu.sync_copy(dst3_hbm.at[wid], idx_v)
        plsc.subcore_barrier()

        def body(j, _):
            pltpu.sync_copy(scaled_hbm.at[pl.ds(base + j * _CH, _CH)], rows_v)
            pltpu.sync_copy(rows_v, acc_sh.at[idx_v.at[j]], add=True)
            return 0

        lax.fori_loop(0, n_ch, body, 0)
        plsc.subcore_barrier()
        pltpu.sync_copy(acc_sh.at[pl.ds(row0, rows_per_tile)],
                        out_hbm.at[cid, pl.ds(row0, rows_per_tile)])

    return k(scaled, dst3, zeros)


def _edge_block_kernel(he_ref, hs_ref, hd_ref, fe_ref, fes_ref, norm_ref,
                       We1_ref, be1_ref, We2_ref, Wf1_ref, Wf2_ref,
                       Wu1_ref, bu1_ref, Wu2_ref, S_ref,
                       he_out_ref, scaled_ref):
    he = he_ref[...]
    hs = hs_ref[...]
    hd = hd_ref[...]
    x = jnp.concatenate([he, hs, hd], axis=1)
    a = jnp.dot(x, We1_ref[...], preferred_element_type=jnp.float32) + be1_ref[...]
    av = a * jax.nn.sigmoid(a)
    v = jnp.dot(av, We2_ref[...], preferred_element_type=jnp.float32)
    r = jnp.maximum(
        jnp.dot(fes_ref[...], Wf1_ref[...], preferred_element_type=jnp.float32)
        * (1.0 / np.sqrt(_EB)), 0.0)
    w = jnp.dot(r, Wf2_ref[...], preferred_element_type=jnp.float32) \
        * (1.0 / np.sqrt(_FCH))
    d = jnp.dot(v * fe_ref[...], S_ref[...],
                preferred_element_type=jnp.float32)
    tp = (w[:, 0:_D] * d[:, 0:1]
          + w[:, _D:2 * _D] * d[:, 1:2]
          + w[:, 2 * _D:3 * _D] * d[:, 2:3])
    u = jnp.concatenate([tp, hs, hd], axis=1)
    b = jnp.dot(u, Wu1_ref[...], preferred_element_type=jnp.float32) + bu1_ref[...]
    bv = b * jax.nn.sigmoid(b)
    he_new = he + jnp.dot(bv, Wu2_ref[...], preferred_element_type=jnp.float32)
    he_out_ref[...] = he_new
    scaled_ref[...] = he_new * norm_ref[...]


def _edge_stage(he, hs, hd, fe, fes, norm2d, We1, be1, We2, Wf1, Wf2,
                Wu1, bu1, Wu2, S):
    E = he.shape[0]
    BE = 2560 if E % 2560 == 0 else 512
    grid = (E // BE,)
    full = lambda shape: pl.BlockSpec(shape, lambda i: (0, 0))
    blk = lambda w: pl.BlockSpec((BE, w), lambda i: (i, 0))
    return pl.pallas_call(
        _edge_block_kernel,
        grid=grid,
        in_specs=[
            blk(_D), blk(_D), blk(_D), blk(_DSH), blk(_EB), blk(1),
            full(We1.shape), full((1, _HID)), full(We2.shape),
            full(Wf1.shape), full(Wf2.shape),
            full(Wu1.shape), full((1, _HID)), full(Wu2.shape),
            full(S.shape),
        ],
        out_specs=[blk(_D), blk(_D)],
        out_shape=[
            jax.ShapeDtypeStruct((E, _D), jnp.float32),
            jax.ShapeDtypeStruct((E, _D), jnp.float32),
        ],
    )(he, hs, hd, fe, fes, norm2d, We1, be1.reshape(1, -1), We2, Wf1, Wf2,
      Wu1, bu1.reshape(1, -1), Wu2, S)


def _node_block_kernel(hn_ref, p0_ref, p1_ref, Wn1_ref, bn1_ref, Wn2_ref,
                       out_ref):
    hn = hn_ref[...]
    x = jnp.concatenate([hn, p0_ref[...] + p1_ref[...]], axis=1)
    a = jnp.dot(x, Wn1_ref[...], preferred_element_type=jnp.float32) + bn1_ref[...]
    av = a * jax.nn.sigmoid(a)
    out_ref[...] = hn + jnp.dot(av, Wn2_ref[...], preferred_element_type=jnp.float32)


def _node_stage(hn, p0, p1, Wn1, bn1, Wn2):
    N = hn.shape[0]
    BN = 2000 if N % 2000 == 0 else N
    grid = (N // BN,)
    full = lambda shape: pl.BlockSpec(shape, lambda i: (0, 0))
    blk = lambda w: pl.BlockSpec((BN, w), lambda i: (i, 0))
    return pl.pallas_call(
        _node_block_kernel,
        grid=grid,
        in_specs=[blk(_D), blk(_D), blk(_D), full(Wn1.shape), full((1, _HID)),
                  full(Wn2.shape)],
        out_specs=blk(_D),
        out_shape=jax.ShapeDtypeStruct((N, _D), jnp.float32),
    )(hn, p0, p1, Wn1, bn1.reshape(1, -1), Wn2)


def kernel(hn, he, fe, fes, norm, edge_index, We1, be1, We2, Wf1, Wf2,
           Wu1, bu1, Wu2, Wn1, bn1, Wn2):
    src = edge_index[0]
    dst = edge_index[1]
    S = np.zeros((_DSH, 3), dtype=np.float32)
    S[0, 0] = 1.0
    S[1:4, 1] = 1.0 / np.sqrt(3.0)
    S[4:9, 2] = 1.0 / np.sqrt(5.0)
    S = jnp.asarray(S / np.sqrt(3.0))

    hs, hd = _gather_stage(hn, src, dst)

    he_new, scaled = _edge_stage(he, hs, hd, fe, fes, norm.reshape(-1, 1),
                                 We1, be1, We2, Wf1, Wf2, Wu1, bu1, Wu2, S)

    N = hn.shape[0]
    E = he.shape[0]
    N_pad = ((N + 8 * _NS - 1) // (8 * _NS)) * (8 * _NS)
    dst3 = dst.reshape(_NW, (E // _NW) // _CH, _CH)
    zeros = jnp.zeros((N_pad, _D), dtype=jnp.float32)
    partials = _scatter_stage(scaled, dst3, zeros, N_pad)

    hn_new = _node_stage(hn, partials[0, :N], partials[1, :N], Wn1, bn1, Wn2)
    return hn_new, he_new


# speedup vs baseline: 2.9007x; 1.0217x over previous
import functools

import numpy as np
import jax
import jax.numpy as jnp
from jax import lax
from jax.experimental import pallas as pl
from jax.experimental.pallas import tpu as pltpu
from jax.experimental.pallas import tpu_sc as plsc

_NC = 2
_NS = 16
_NW = _NC * _NS

_D = 128
_DSH = 9
_EB = 16
_FCH = 16
_HID = 4 * _D


_CH = 80


def _gather_stage(hn, src, dst):
    N, D = hn.shape
    E = src.shape[0]
    per_w = E // _NW
    n_ch = per_w // _CH
    assert per_w * _NW == E and n_ch * _CH == per_w
    mesh = plsc.VectorSubcoreMesh(core_axis_name="c", subcore_axis_name="s")

    n_pairs = (n_ch - 1) // 2
    has_tail = (n_ch % 2) == 1

    def k(hn_hbm, src_hbm, dst_hbm, hs_hbm, hd_hbm,
          idx_v, buf0, buf1, gsem0, gsem1):
        wid = lax.axis_index("s") * _NC + lax.axis_index("c")
        base = wid * per_w

        def gather_chunk(c, buf, sem):
            return pltpu.async_copy(
                hn_hbm.at[idx_v.at[pl.ds(c * _CH, _CH)]], buf, sem)

        for idx_hbm, out_hbm in ((src_hbm, hs_hbm), (dst_hbm, hd_hbm)):
            pltpu.sync_copy(idx_hbm.at[pl.ds(base, per_w)], idx_v)
            gather_chunk(0, buf0, gsem0)

            def body(i, _, out_hbm=out_hbm):
                c = 2 * i
                pltpu.make_async_copy(hn_hbm.at[idx_v.at[pl.ds(0, _CH)]],
                                      buf0, gsem0).wait()
                gather_chunk(c + 1, buf1, gsem1)
                pltpu.sync_copy(buf0, out_hbm.at[pl.ds(base + c * _CH, _CH)])
                pltpu.make_async_copy(hn_hbm.at[idx_v.at[pl.ds(0, _CH)]],
                                      buf1, gsem1).wait()
                gather_chunk(c + 2, buf0, gsem0)
                pltpu.sync_copy(buf1,
                                out_hbm.at[pl.ds(base + (c + 1) * _CH, _CH)])
                return 0

            lax.fori_loop(0, n_pairs, body, 0)
            if has_tail:
                pltpu.make_async_copy(hn_hbm.at[idx_v.at[pl.ds(0, _CH)]],
                                      buf0, gsem0).wait()
                pltpu.sync_copy(
                    buf0, out_hbm.at[pl.ds(base + (n_ch - 1) * _CH, _CH)])

    return pl.kernel(
        k,
        out_type=[jax.ShapeDtypeStruct((E, D), jnp.float32),
                  jax.ShapeDtypeStruct((E, D), jnp.float32)],
        mesh=mesh,
        scratch_types=[
            pltpu.VMEM((per_w,), jnp.int32),
            pltpu.VMEM((_CH, D), jnp.float32),
            pltpu.VMEM((_CH, D), jnp.float32),
            pltpu.SemaphoreType.DMA,
            pltpu.SemaphoreType.DMA,
        ],
    )(hn, src, dst)


def _scatter_stage(scaled, dst3, zeros, N_pad):
    E, D = scaled.shape
    per_w = E // _NW
    n_ch = per_w // _CH
    rows_per_tile = N_pad // _NS
    assert rows_per_tile * _NS == N_pad and rows_per_tile % 8 == 0
    mesh = plsc.VectorSubcoreMesh(core_axis_name="c", subcore_axis_name="s")

    @functools.partial(
        pl.kernel, mesh=mesh,
        out_type=jax.ShapeDtypeStruct((_NC, N_pad, D), jnp.float32),
        scratch_types=[
            pltpu.VMEM((n_ch, _CH), jnp.int32),
            pltpu.VMEM((_CH, D), jnp.float32),
            pltpu.VMEM_SHARED((N_pad, D), jnp.float32),
        ],
    )
    def k(scaled_hbm, dst3_hbm, zeros_hbm, out_hbm, idx_v, rows_v, acc_sh):
        cid = lax.axis_index("c")
        sid = lax.axis_index("s")
        wid = sid * _NC + cid
        base = wid * per_w
        row0 = sid * rows_per_tile
        pltpu.sync_copy(zeros_hbm.at[pl.ds(row0, rows_per_tile)],
                        acc_sh.at[pl.ds(row0, rows_per_tile)])
        pltpu.sync_copy(dst3_hbm.at[wid], idx_v)
        plsc.subcore_barrier()

        def body(j, _):
            pltpu.sync_copy(scaled_hbm.at[pl.ds(base + j * _CH, _CH)], rows_v)
            pltpu.sync_copy(rows_v, acc_sh.at[idx_v.at[j]], add=True)
            return 0

        lax.fori_loop(0, n_ch, body, 0)
        plsc.subcore_barrier()
        pltpu.sync_copy(acc_sh.at[pl.ds(row0, rows_per_tile)],
                        out_hbm.at[cid, pl.ds(row0, rows_per_tile)])

    return k(scaled, dst3, zeros)


def _edge_block_kernel(he_ref, hs_ref, hd_ref, fe_ref, fes_ref, norm_ref,
                       We1_ref, be1_ref, We2_ref, Wf1_ref, Wf2_ref,
                       Wu1_ref, bu1_ref, Wu2_ref, S_ref,
                       he_out_ref, scaled_ref):
    bf = jnp.bfloat16
    he = he_ref[...]
    hs16 = hs_ref[...].astype(bf)
    hd16 = hd_ref[...].astype(bf)
    x = jnp.concatenate([he.astype(bf), hs16, hd16], axis=1)
    a = jnp.dot(x, We1_ref[...].astype(bf),
                preferred_element_type=jnp.float32) + be1_ref[...]
    av = (a * jax.nn.sigmoid(a)).astype(bf)
    v = jnp.dot(av, We2_ref[...].astype(bf),
                preferred_element_type=jnp.float32)
    r = jnp.maximum(
        jnp.dot(fes_ref[...], Wf1_ref[...], preferred_element_type=jnp.float32)
        * (1.0 / np.sqrt(_EB)), 0.0)
    w = jnp.dot(r, Wf2_ref[...], preferred_element_type=jnp.float32) \
        * (1.0 / np.sqrt(_FCH))
    d = jnp.dot(v * fe_ref[...], S_ref[...],
                preferred_element_type=jnp.float32)
    tp = (w[:, 0:_D] * d[:, 0:1]
          + w[:, _D:2 * _D] * d[:, 1:2]
          + w[:, 2 * _D:3 * _D] * d[:, 2:3])
    u = jnp.concatenate([tp.astype(bf), hs16, hd16], axis=1)
    b = jnp.dot(u, Wu1_ref[...].astype(bf),
                preferred_element_type=jnp.float32) + bu1_ref[...]
    bv = (b * jax.nn.sigmoid(b)).astype(bf)
    he_new = he + jnp.dot(bv, Wu2_ref[...].astype(bf),
                          preferred_element_type=jnp.float32)
    he_out_ref[...] = he_new
    scaled_ref[...] = he_new * norm_ref[...]


def _edge_stage(he, hs, hd, fe, fes, norm2d, We1, be1, We2, Wf1, Wf2,
                Wu1, bu1, Wu2, S):
    E = he.shape[0]
    BE = 2560 if E % 2560 == 0 else 512
    grid = (E // BE,)
    full = lambda shape: pl.BlockSpec(shape, lambda i: (0, 0))
    blk = lambda w: pl.BlockSpec((BE, w), lambda i: (i, 0))
    return pl.pallas_call(
        _edge_block_kernel,
        grid=grid,
        in_specs=[
            blk(_D), blk(_D), blk(_D), blk(_DSH), blk(_EB), blk(1),
            full(We1.shape), full((1, _HID)), full(We2.shape),
            full(Wf1.shape), full(Wf2.shape),
            full(Wu1.shape), full((1, _HID)), full(Wu2.shape),
            full(S.shape),
        ],
        out_specs=[blk(_D), blk(_D)],
        out_shape=[
            jax.ShapeDtypeStruct((E, _D), jnp.float32),
            jax.ShapeDtypeStruct((E, _D), jnp.float32),
        ],
    )(he, hs, hd, fe, fes, norm2d, We1, be1.reshape(1, -1), We2, Wf1, Wf2,
      Wu1, bu1.reshape(1, -1), Wu2, S)


def _node_block_kernel(hn_ref, p0_ref, p1_ref, Wn1_ref, bn1_ref, Wn2_ref,
                       out_ref):
    hn = hn_ref[...]
    x = jnp.concatenate([hn, p0_ref[...] + p1_ref[...]], axis=1)
    a = jnp.dot(x, Wn1_ref[...], preferred_element_type=jnp.float32) + bn1_ref[...]
    av = a * jax.nn.sigmoid(a)
    out_ref[...] = hn + jnp.dot(av, Wn2_ref[...], preferred_element_type=jnp.float32)


def _node_stage(hn, p0, p1, Wn1, bn1, Wn2):
    N = hn.shape[0]
    BN = 2000 if N % 2000 == 0 else N
    grid = (N // BN,)
    full = lambda shape: pl.BlockSpec(shape, lambda i: (0, 0))
    blk = lambda w: pl.BlockSpec((BN, w), lambda i: (i, 0))
    return pl.pallas_call(
        _node_block_kernel,
        grid=grid,
        in_specs=[blk(_D), blk(_D), blk(_D), full(Wn1.shape), full((1, _HID)),
                  full(Wn2.shape)],
        out_specs=blk(_D),
        out_shape=jax.ShapeDtypeStruct((N, _D), jnp.float32),
    )(hn, p0, p1, Wn1, bn1.reshape(1, -1), Wn2)


def kernel(hn, he, fe, fes, norm, edge_index, We1, be1, We2, Wf1, Wf2,
           Wu1, bu1, Wu2, Wn1, bn1, Wn2):
    src = edge_index[0]
    dst = edge_index[1]
    S = np.zeros((_DSH, 3), dtype=np.float32)
    S[0, 0] = 1.0
    S[1:4, 1] = 1.0 / np.sqrt(3.0)
    S[4:9, 2] = 1.0 / np.sqrt(5.0)
    S = jnp.asarray(S / np.sqrt(3.0))

    hs, hd = _gather_stage(hn, src, dst)

    he_new, scaled = _edge_stage(he, hs, hd, fe, fes, norm.reshape(-1, 1),
                                 We1, be1, We2, Wf1, Wf2, Wu1, bu1, Wu2, S)

    N = hn.shape[0]
    E = he.shape[0]
    N_pad = ((N + 8 * _NS - 1) // (8 * _NS)) * (8 * _NS)
    dst3 = dst.reshape(_NW, (E // _NW) // _CH, _CH)
    zeros = jnp.zeros((N_pad, _D), dtype=jnp.float32)
    partials = _scatter_stage(scaled, dst3, zeros, N_pad)

    hn_new = _node_stage(hn, partials[0, :N], partials[1, :N], Wn1, bn1, Wn2)
    return hn_new, he_new


# speedup vs baseline: 3.2025x; 1.1041x over previous
import functools

import numpy as np
import jax
import jax.numpy as jnp
from jax import lax
from jax.experimental import pallas as pl
from jax.experimental.pallas import tpu as pltpu
from jax.experimental.pallas import tpu_sc as plsc

_NC = 2
_NS = 16
_NW = _NC * _NS

_D = 128
_DSH = 9
_EB = 16
_FCH = 16
_HID = 4 * _D


_CH = 80


def _gather_stage(hn, src, dst):
    N, D = hn.shape
    E = src.shape[0]
    per_w = E // _NW
    n_ch = per_w // _CH
    assert per_w * _NW == E and n_ch * _CH == per_w
    mesh = plsc.VectorSubcoreMesh(core_axis_name="c", subcore_axis_name="s")

    n_pairs = (n_ch - 1) // 2
    has_tail = (n_ch % 2) == 1

    def k(hn_hbm, src_hbm, dst_hbm, hs_hbm, hd_hbm,
          idx_v, buf0, buf1, gsem0, gsem1):
        wid = lax.axis_index("s") * _NC + lax.axis_index("c")
        base = wid * per_w

        def gather_chunk(c, buf, sem):
            return pltpu.async_copy(
                hn_hbm.at[idx_v.at[pl.ds(c * _CH, _CH)]], buf, sem)

        for idx_hbm, out_hbm in ((src_hbm, hs_hbm), (dst_hbm, hd_hbm)):
            pltpu.sync_copy(idx_hbm.at[pl.ds(base, per_w)], idx_v)
            gather_chunk(0, buf0, gsem0)

            def body(i, _, out_hbm=out_hbm):
                c = 2 * i
                pltpu.make_async_copy(hn_hbm.at[idx_v.at[pl.ds(0, _CH)]],
                                      buf0, gsem0).wait()
                gather_chunk(c + 1, buf1, gsem1)
                pltpu.sync_copy(buf0, out_hbm.at[pl.ds(base + c * _CH, _CH)])
                pltpu.make_async_copy(hn_hbm.at[idx_v.at[pl.ds(0, _CH)]],
                                      buf1, gsem1).wait()
                gather_chunk(c + 2, buf0, gsem0)
                pltpu.sync_copy(buf1,
                                out_hbm.at[pl.ds(base + (c + 1) * _CH, _CH)])
                return 0

            lax.fori_loop(0, n_pairs, body, 0)
            if has_tail:
                pltpu.make_async_copy(hn_hbm.at[idx_v.at[pl.ds(0, _CH)]],
                                      buf0, gsem0).wait()
                pltpu.sync_copy(
                    buf0, out_hbm.at[pl.ds(base + (n_ch - 1) * _CH, _CH)])

    return pl.kernel(
        k,
        out_type=[jax.ShapeDtypeStruct((E, D), jnp.float32),
                  jax.ShapeDtypeStruct((E, D), jnp.float32)],
        mesh=mesh,
        scratch_types=[
            pltpu.VMEM((per_w,), jnp.int32),
            pltpu.VMEM((_CH, D), jnp.float32),
            pltpu.VMEM((_CH, D), jnp.float32),
            pltpu.SemaphoreType.DMA,
            pltpu.SemaphoreType.DMA,
        ],
    )(hn, src, dst)


def _scatter_stage(scaled, dst3, zeros, N_pad):
    E, D = scaled.shape
    per_w = E // _NW
    n_ch = per_w // _CH
    rows_per_tile = N_pad // _NS
    assert rows_per_tile * _NS == N_pad and rows_per_tile % 8 == 0
    mesh = plsc.VectorSubcoreMesh(core_axis_name="c", subcore_axis_name="s")

    @functools.partial(
        pl.kernel, mesh=mesh,
        out_type=jax.ShapeDtypeStruct((_NC, N_pad, D), jnp.float32),
        scratch_types=[
            pltpu.VMEM((n_ch, _CH), jnp.int32),
            pltpu.VMEM((_CH, D), jnp.float32),
            pltpu.VMEM_SHARED((N_pad, D), jnp.float32),
        ],
    )
    def k(scaled_hbm, dst3_hbm, zeros_hbm, out_hbm, idx_v, rows_v, acc_sh):
        cid = lax.axis_index("c")
        sid = lax.axis_index("s")
        wid = sid * _NC + cid
        base = wid * per_w
        row0 = sid * rows_per_tile
        pltpu.sync_copy(zeros_hbm.at[pl.ds(row0, rows_per_tile)],
                        acc_sh.at[pl.ds(row0, rows_per_tile)])
        pltpu.sync_copy(dst3_hbm.at[wid], idx_v)
        plsc.subcore_barrier()

        def body(j, _):
            pltpu.sync_copy(scaled_hbm.at[pl.ds(base + j * _CH, _CH)], rows_v)
            pltpu.sync_copy(rows_v, acc_sh.at[idx_v.at[j]], add=True)
            return 0

        lax.fori_loop(0, n_ch, body, 0)
        plsc.subcore_barrier()
        pltpu.sync_copy(acc_sh.at[pl.ds(row0, rows_per_tile)],
                        out_hbm.at[cid, pl.ds(row0, rows_per_tile)])

    return k(scaled, dst3, zeros)


def _edge_block_kernel(he_ref, hs_ref, hd_ref, fe_ref, fes_ref, norm_ref,
                       We1_ref, be1_ref, We2_ref, Wf1_ref, Wf2_ref,
                       Wu1_ref, bu1_ref, Wu2_ref, SR_ref,
                       he_out_ref, scaled_ref):
    bf = jnp.bfloat16
    he = he_ref[...]
    hs16 = hs_ref[...].astype(bf)
    hd16 = hd_ref[...].astype(bf)
    x = jnp.concatenate([he.astype(bf), hs16, hd16], axis=1)
    a = jnp.dot(x, We1_ref[...].astype(bf),
                preferred_element_type=jnp.float32) + be1_ref[...]
    ha = 0.5 * a
    av = (ha * (1.0 + jnp.tanh(ha))).astype(bf)
    v = jnp.dot(av, We2_ref[...].astype(bf),
                preferred_element_type=jnp.float32)
    r = jnp.maximum(
        jnp.dot(fes_ref[...], Wf1_ref[...], preferred_element_type=jnp.float32)
        * (1.0 / np.sqrt(_EB)), 0.0)
    w = jnp.dot(r, Wf2_ref[...], preferred_element_type=jnp.float32) \
        * (1.0 / np.sqrt(_FCH))
    dwide = jnp.dot(v * fe_ref[...], SR_ref[...],
                    preferred_element_type=jnp.float32)
    wd = w * dwide
    tp = wd[:, 0:_D] + wd[:, _D:2 * _D] + wd[:, 2 * _D:3 * _D]
    u = jnp.concatenate([tp.astype(bf), hs16, hd16], axis=1)
    b = jnp.dot(u, Wu1_ref[...].astype(bf),
                preferred_element_type=jnp.float32) + bu1_ref[...]
    hb = 0.5 * b
    bv = (hb * (1.0 + jnp.tanh(hb))).astype(bf)
    he_new = he + jnp.dot(bv, Wu2_ref[...].astype(bf),
                          preferred_element_type=jnp.float32)
    he_out_ref[...] = he_new
    scaled_ref[...] = he_new * norm_ref[...]


def _edge_stage(he, hs, hd, fe, fes, norm2d, We1, be1, We2, Wf1, Wf2,
                Wu1, bu1, Wu2, SR):
    E = he.shape[0]
    BE = 2560 if E % 2560 == 0 else 512
    grid = (E // BE,)
    full = lambda shape: pl.BlockSpec(shape, lambda i: (0, 0))
    blk = lambda w: pl.BlockSpec((BE, w), lambda i: (i, 0))
    return pl.pallas_call(
        _edge_block_kernel,
        grid=grid,
        in_specs=[
            blk(_D), blk(_D), blk(_D), blk(_DSH), blk(_EB), blk(1),
            full(We1.shape), full((1, _HID)), full(We2.shape),
            full(Wf1.shape), full(Wf2.shape),
            full(Wu1.shape), full((1, _HID)), full(Wu2.shape),
            full(SR.shape),
        ],
        out_specs=[blk(_D), blk(_D)],
        out_shape=[
            jax.ShapeDtypeStruct((E, _D), jnp.float32),
            jax.ShapeDtypeStruct((E, _D), jnp.float32),
        ],
    )(he, hs, hd, fe, fes, norm2d, We1, be1.reshape(1, -1), We2, Wf1, Wf2,
      Wu1, bu1.reshape(1, -1), Wu2, SR)


def _node_block_kernel(hn_ref, p0_ref, p1_ref, Wn1_ref, bn1_ref, Wn2_ref,
                       out_ref):
    hn = hn_ref[...]
    x = jnp.concatenate([hn, p0_ref[...] + p1_ref[...]], axis=1)
    a = jnp.dot(x, Wn1_ref[...], preferred_element_type=jnp.float32) + bn1_ref[...]
    ha = 0.5 * a
    av = ha * (1.0 + jnp.tanh(ha))
    out_ref[...] = hn + jnp.dot(av, Wn2_ref[...], preferred_element_type=jnp.float32)


def _node_stage(hn, p0, p1, Wn1, bn1, Wn2):
    N = hn.shape[0]
    BN = 2000 if N % 2000 == 0 else N
    grid = (N // BN,)
    full = lambda shape: pl.BlockSpec(shape, lambda i: (0, 0))
    blk = lambda w: pl.BlockSpec((BN, w), lambda i: (i, 0))
    return pl.pallas_call(
        _node_block_kernel,
        grid=grid,
        in_specs=[blk(_D), blk(_D), blk(_D), full(Wn1.shape), full((1, _HID)),
                  full(Wn2.shape)],
        out_specs=blk(_D),
        out_shape=jax.ShapeDtypeStruct((N, _D), jnp.float32),
    )(hn, p0, p1, Wn1, bn1.reshape(1, -1), Wn2)


def kernel(hn, he, fe, fes, norm, edge_index, We1, be1, We2, Wf1, Wf2,
           Wu1, bu1, Wu2, Wn1, bn1, Wn2):
    src = edge_index[0]
    dst = edge_index[1]
    S = np.zeros((_DSH, 3), dtype=np.float32)
    S[0, 0] = 1.0
    S[1:4, 1] = 1.0 / np.sqrt(3.0)
    S[4:9, 2] = 1.0 / np.sqrt(5.0)
    S = S / np.sqrt(3.0)
    SR = jnp.asarray(np.repeat(S, _D, axis=1))

    hs, hd = _gather_stage(hn, src, dst)

    he_new, scaled = _edge_stage(he, hs, hd, fe, fes, norm.reshape(-1, 1),
                                 We1, be1, We2, Wf1, Wf2, Wu1, bu1, Wu2, SR)

    N = hn.shape[0]
    E = he.shape[0]
    N_pad = ((N + 8 * _NS - 1) // (8 * _NS)) * (8 * _NS)
    dst3 = dst.reshape(_NW, (E // _NW) // _CH, _CH)
    zeros = jnp.zeros((N_pad, _D), dtype=jnp.float32)
    partials = _scatter_stage(scaled, dst3, zeros, N_pad)

    hn_new = _node_stage(hn, partials[0, :N], partials[1, :N], Wn1, bn1, Wn2)
    return hn_new, he_new
